# baseline (speedup 1.0000x reference)
import functools

import jax
import jax.numpy as jnp
from jax import lax
from jax.experimental import pallas as pl
from jax.experimental.pallas import tpu as pltpu
from jax.experimental.pallas import tpu_sc as plsc

_LANES = 128
_D = 65
_DP = 128
_P1_CHUNK = 200
_P2_CHUNK = 128

_COL_OFFS = (0, 16, 32, 48, 49)


def _repack(dst, src, n_rows):
    unroll = 8
    assert n_rows % unroll == 0

    def rows(i, c):
        for u in range(unroll):
            r = i * unroll + u
            for j in _COL_OFFS:
                dst[r, pl.ds(j, 16)] = src[r, pl.ds(j, 16)]
        return c

    lax.fori_loop(0, n_rows // unroll, rows, 0)


@functools.cache
def _make_phase1(n_rows):
    info = plsc.get_sparse_core_info()
    nw = info.num_cores * info.num_subcores
    n_chunks = n_rows // _P1_CHUNK
    n_iters = -(-n_chunks // (2 * nw))
    mesh = plsc.VectorSubcoreMesh(core_axis_name="c", subcore_axis_name="s")

    @functools.partial(
        pl.kernel,
        mesh=mesh,
        out_type=jax.ShapeDtypeStruct((n_rows, _DP), jnp.float32),
        scratch_types=[
            pltpu.VMEM((_P1_CHUNK, _D), jnp.float32),
            pltpu.VMEM((_P1_CHUNK, _D), jnp.float32),
            pltpu.VMEM((_P1_CHUNK, _DP), jnp.float32),
            pltpu.VMEM((_P1_CHUNK, _DP), jnp.float32),
            pltpu.SemaphoreType.DMA,
            pltpu.SemaphoreType.DMA,
            pltpu.SemaphoreType.DMA,
            pltpu.SemaphoreType.DMA,
        ],
    )
    def phase1(table_hbm, padded_hbm, b65_0, b65_1, b128_0, b128_1,
               si0, si1, so0, so1):
        wid = lax.axis_index("s") * info.num_cores + lax.axis_index("c")
        b65 = (b65_0, b65_1)
        b128 = (b128_0, b128_1)
        sin = (si0, si1)
        sout = (so0, so1)

        def in_start(s, g):
            pltpu.async_copy(
                table_hbm.at[pl.ds(g * _P1_CHUNK, _P1_CHUNK)], b65[s], sin[s])

        def in_wait(s):
            pltpu.make_async_copy(
                table_hbm.at[pl.ds(0, _P1_CHUNK)], b65[s], sin[s]).wait()

        def out_start(s, g):
            pltpu.async_copy(
                b128[s], padded_hbm.at[pl.ds(g * _P1_CHUNK, _P1_CHUNK)],
                sout[s])

        def out_wait(s):
            pltpu.make_async_copy(
                b128[s], padded_hbm.at[pl.ds(0, _P1_CHUNK)], sout[s]).wait()

        for s in range(2):
            g = wid + 32 * s

            @pl.when(g < n_chunks)
            def _(s=s, g=g):
                in_start(s, g)

        def body(t, carry):
            for s in range(2):
                g = wid + 64 * t + 32 * s

                @pl.when(g < n_chunks)
                def _(s=s, g=g):
                    in_wait(s)

                    @pl.when(t >= 1)
                    def _():
                        out_wait(s)

                    _repack(b128[s], b65[s], _P1_CHUNK)
                    out_start(s, g)

                    @pl.when(g + 64 < n_chunks)
                    def _():
                        in_start(s, g + 64)

            return carry

        lax.fori_loop(0, n_iters, body, 0)
        for s in range(2):
            out_wait(s)

    return phase1


@functools.cache
def _make_phase2(total, n_rows):
    info = plsc.get_sparse_core_info()
    nw = info.num_cores * info.num_subcores
    per_w = total // nw
    n_chunks = per_w // _P2_CHUNK
    mesh = plsc.VectorSubcoreMesh(core_axis_name="c", subcore_axis_name="s")

    @functools.partial(
        pl.kernel,
        mesh=mesh,
        out_type=jax.ShapeDtypeStruct((total, _D), jnp.float32),
        scratch_types=[
            pltpu.VMEM((per_w,), jnp.int32),
            pltpu.VMEM((_P2_CHUNK, _DP), jnp.float32),
            pltpu.VMEM((_P2_CHUNK, _DP), jnp.float32),
            pltpu.VMEM((_P2_CHUNK, _D), jnp.float32),
            pltpu.VMEM((_P2_CHUNK, _D), jnp.float32),
            pltpu.SemaphoreType.DMA,
            pltpu.SemaphoreType.DMA,
            pltpu.SemaphoreType.DMA,
            pltpu.SemaphoreType.DMA,
        ],
    )
    def phase2(idx_hbm, padded_hbm, out_hbm, idx_v, r128_0, r128_1,
               r65_0, r65_1, sg0, sg1, so0, so1):
        wid = lax.axis_index("s") * info.num_cores + lax.axis_index("c")
        base = wid * per_w
        r128 = (r128_0, r128_1)
        r65 = (r65_0, r65_1)
        sg = (sg0, sg1)
        so = (so0, so1)

        pltpu.sync_copy(idx_hbm.at[pl.ds(base, per_w)], idx_v)

        def g_start(s, k):
            pltpu.async_copy(
                padded_hbm.at[idx_v.at[pl.ds(k * _P2_CHUNK, _P2_CHUNK)]],
                r128[s], sg[s])

        def g_wait(s):
            pltpu.make_async_copy(
                padded_hbm.at[pl.ds(0, _P2_CHUNK)], r128[s], sg[s]).wait()

        def o_start(s, k):
            pltpu.async_copy(
                r65[s],
                out_hbm.at[pl.ds(base + k * _P2_CHUNK, _P2_CHUNK)], so[s])

        def o_wait(s):
            pltpu.make_async_copy(
                r65[s], out_hbm.at[pl.ds(0, _P2_CHUNK)], so[s]).wait()

        for s in range(2):
            g_start(s, s)

        def body(t, carry):
            for s in range(2):
                k = 2 * t + s
                g_wait(s)

                @pl.when(t >= 1)
                def _(s=s):
                    o_wait(s)

                _repack(r65[s], r128[s], _P2_CHUNK)
                o_start(s, k)

                @pl.when(k + 2 < n_chunks)
                def _(s=s, k=k):
                    g_start(s, k + 2)

            return carry

        lax.fori_loop(0, n_chunks // 2, body, 0)
        for s in range(2):
            o_wait(s)

    return phase2


def kernel(x, table):
    b, h = x.shape
    total = b * h
    n_rows = table.shape[0]
    idx = jnp.maximum(x.reshape(-1).astype(jnp.int32), 0)
    padded = _make_phase1(n_rows)(table)
    out = _make_phase2(total, n_rows)(idx, padded)
    return out.reshape(b, h, table.shape[1])


# speedup vs baseline: 1.0014x; 1.0014x over previous
import functools

import jax
import jax.numpy as jnp
from jax import lax
from jax.experimental import pallas as pl
from jax.experimental.pallas import tpu as pltpu
from jax.experimental.pallas import tpu_sc as plsc

_LANES = 128
_D = 65
_DP = 128
_P1_CHUNK = 200
_P2_CHUNK = 128

_COL_OFFS = (0, 16, 32, 48, 49)


def _repack(dst, src, n_rows):
    unroll = 8
    assert n_rows % unroll == 0

    def rows(i, c):
        for u in range(unroll):
            r = i * unroll + u
            for j in _COL_OFFS:
                dst[r, pl.ds(j, 16)] = src[r, pl.ds(j, 16)]
        return c

    lax.fori_loop(0, n_rows // unroll, rows, 0)


@functools.cache
def _make_phase1(n_rows):
    info = plsc.get_sparse_core_info()
    nw = info.num_cores * info.num_subcores
    n_chunks = n_rows // _P1_CHUNK
    n_iters = -(-n_chunks // (2 * nw))
    mesh = plsc.VectorSubcoreMesh(core_axis_name="c", subcore_axis_name="s")

    @functools.partial(
        pl.kernel,
        mesh=mesh,
        out_type=jax.ShapeDtypeStruct((n_rows, _DP), jnp.float32),
        compiler_params=pltpu.CompilerParams(skip_device_barrier=True),
        scratch_types=[
            pltpu.VMEM((_P1_CHUNK, _D), jnp.float32),
            pltpu.VMEM((_P1_CHUNK, _D), jnp.float32),
            pltpu.VMEM((_P1_CHUNK, _DP), jnp.float32),
            pltpu.VMEM((_P1_CHUNK, _DP), jnp.float32),
            pltpu.SemaphoreType.DMA,
            pltpu.SemaphoreType.DMA,
            pltpu.SemaphoreType.DMA,
            pltpu.SemaphoreType.DMA,
        ],
    )
    def phase1(table_hbm, padded_hbm, b65_0, b65_1, b128_0, b128_1,
               si0, si1, so0, so1):
        wid = lax.axis_index("s") * info.num_cores + lax.axis_index("c")
        b65 = (b65_0, b65_1)
        b128 = (b128_0, b128_1)
        sin = (si0, si1)
        sout = (so0, so1)

        def in_start(s, g):
            pltpu.async_copy(
                table_hbm.at[pl.ds(g * _P1_CHUNK, _P1_CHUNK)], b65[s], sin[s])

        def in_wait(s):
            pltpu.make_async_copy(
                table_hbm.at[pl.ds(0, _P1_CHUNK)], b65[s], sin[s]).wait()

        def out_start(s, g):
            pltpu.async_copy(
                b128[s], padded_hbm.at[pl.ds(g * _P1_CHUNK, _P1_CHUNK)],
                sout[s])

        def out_wait(s):
            pltpu.make_async_copy(
                b128[s], padded_hbm.at[pl.ds(0, _P1_CHUNK)], sout[s]).wait()

        for s in range(2):
            g = wid + 32 * s

            @pl.when(g < n_chunks)
            def _(s=s, g=g):
                in_start(s, g)

        def body(t, carry):
            for s in range(2):
                g = wid + 64 * t + 32 * s

                @pl.when(g < n_chunks)
                def _(s=s, g=g):
                    in_wait(s)

                    @pl.when(t >= 1)
                    def _():
                        out_wait(s)

                    _repack(b128[s], b65[s], _P1_CHUNK)
                    out_start(s, g)

                    @pl.when(g + 64 < n_chunks)
                    def _():
                        in_start(s, g + 64)

            return carry

        lax.fori_loop(0, n_iters, body, 0)
        for s in range(2):
            out_wait(s)

    return phase1


@functools.cache
def _make_phase2(total, n_rows):
    info = plsc.get_sparse_core_info()
    nw = info.num_cores * info.num_subcores
    per_w = total // nw
    n_chunks = per_w // _P2_CHUNK
    mesh = plsc.VectorSubcoreMesh(core_axis_name="c", subcore_axis_name="s")

    @functools.partial(
        pl.kernel,
        mesh=mesh,
        out_type=jax.ShapeDtypeStruct((total, _D), jnp.float32),
        compiler_params=pltpu.CompilerParams(skip_device_barrier=True),
        scratch_types=[
            pltpu.VMEM((per_w,), jnp.int32),
            pltpu.VMEM((_P2_CHUNK, _DP), jnp.float32),
            pltpu.VMEM((_P2_CHUNK, _DP), jnp.float32),
            pltpu.VMEM((_P2_CHUNK, _D), jnp.float32),
            pltpu.VMEM((_P2_CHUNK, _D), jnp.float32),
            pltpu.SemaphoreType.DMA,
            pltpu.SemaphoreType.DMA,
            pltpu.SemaphoreType.DMA,
            pltpu.SemaphoreType.DMA,
        ],
    )
    def phase2(idx_hbm, padded_hbm, out_hbm, idx_v, r128_0, r128_1,
               r65_0, r65_1, sg0, sg1, so0, so1):
        wid = lax.axis_index("s") * info.num_cores + lax.axis_index("c")
        base = wid * per_w
        r128 = (r128_0, r128_1)
        r65 = (r65_0, r65_1)
        sg = (sg0, sg1)
        so = (so0, so1)

        pltpu.sync_copy(idx_hbm.at[pl.ds(base, per_w)], idx_v)

        def g_start(s, k):
            pltpu.async_copy(
                padded_hbm.at[idx_v.at[pl.ds(k * _P2_CHUNK, _P2_CHUNK)]],
                r128[s], sg[s])

        def g_wait(s):
            pltpu.make_async_copy(
                padded_hbm.at[pl.ds(0, _P2_CHUNK)], r128[s], sg[s]).wait()

        def o_start(s, k):
            pltpu.async_copy(
                r65[s],
                out_hbm.at[pl.ds(base + k * _P2_CHUNK, _P2_CHUNK)], so[s])

        def o_wait(s):
            pltpu.make_async_copy(
                r65[s], out_hbm.at[pl.ds(0, _P2_CHUNK)], so[s]).wait()

        for s in range(2):
            g_start(s, s)

        def body(t, carry):
            for s in range(2):
                k = 2 * t + s
                g_wait(s)

                @pl.when(t >= 1)
                def _(s=s):
                    o_wait(s)

                _repack(r65[s], r128[s], _P2_CHUNK)
                o_start(s, k)

                @pl.when(k + 2 < n_chunks)
                def _(s=s, k=k):
                    g_start(s, k + 2)

            return carry

        lax.fori_loop(0, n_chunks // 2, body, 0)
        for s in range(2):
            o_wait(s)

    return phase2


def kernel(x, table):
    b, h = x.shape
    total = b * h
    n_rows = table.shape[0]
    idx = jnp.maximum(x.reshape(-1).astype(jnp.int32), 0)
    padded = _make_phase1(n_rows)(table)
    out = _make_phase2(total, n_rows)(idx, padded)
    return out.reshape(b, h, table.shape[1])
